# baseline (speedup 1.0000x reference)
import functools

import jax
import jax.numpy as jnp
from jax import lax
from jax.experimental import pallas as pl
from jax.experimental.pallas import tpu as pltpu
from jax.experimental.pallas import tpu_sc as plsc

ENT = 1000000
REL = 1000
D = 32
B = 16384
LMBDA = 0.0001

NC = 2
NS = 16
NW = NC * NS
BPW = B // NW
CHUNK = 128
NCH = BPW // CHUNK

_mesh = plsc.VectorSubcoreMesh(core_axis_name="c", subcore_axis_name="s")


@functools.partial(
    pl.kernel,
    mesh=_mesh,
    compiler_params=pltpu.CompilerParams(
        needs_layout_passes=False, use_tc_tiling_on_sc=False),
    out_type=[
        jax.ShapeDtypeStruct((B,), jnp.float32),
        jax.ShapeDtypeStruct((B,), jnp.float32),
        jax.ShapeDtypeStruct((NW, 16), jnp.float32),
    ],
    scratch_types=[
        pltpu.VMEM((NCH, CHUNK), jnp.int32),
        pltpu.VMEM((NCH, CHUNK), jnp.int32),
        pltpu.VMEM((NCH, CHUNK), jnp.int32),
        pltpu.VMEM((BPW, D), jnp.float32),
        pltpu.VMEM((BPW, D), jnp.float32),
        pltpu.VMEM((BPW, D), jnp.float32),
        pltpu.VMEM((BPW, D), jnp.float32),
        pltpu.VMEM((BPW, D), jnp.float32),
        pltpu.VMEM((BPW, D), jnp.float32),
        pltpu.VMEM((BPW,), jnp.float32),
        pltpu.VMEM((16,), jnp.float32),
        pltpu.SemaphoreType.DMA,
    ],
)
def _sc_score(ph, pt, pr, nh, nt, nr, ent1, ent2, rel1, rel2,
              ps_out, ns_out, sq_out,
              idx_h, idx_t, idx_r, e1h, e2h, e1t, e2t, r1v, r2v,
              score_v, sq_v, sem):
    wid = lax.axis_index("s") * NC + lax.axis_index("c")
    rbase = wid * NCH
    sbase = wid * BPW

    def remap(buf):
        for j in range(NCH):
            for k in range(128 // 16):
                e = buf[j, pl.ds(k * 16, 16)]
                q = ((e >= _SEG).astype(jnp.int32)
                     + (e >= 2 * _SEG).astype(jnp.int32)
                     + (e >= 3 * _SEG).astype(jnp.int32))
                g = 4 * (e - q * _SEG) + q
                buf[j, pl.ds(k * 16, 16)] = jnp.where(
                    e >= 4 * _SEG, 4 * e - (12 * _SEG - 3), g)

    total_sq = jnp.zeros((16,), jnp.float32)
    for hh, tt, rr, out_ref in ((ph, pt, pr, ps_out), (nh, nt, nr, ns_out)):
        pltpu.sync_copy(hh.at[pl.ds(rbase, NCH)], idx_h)
        pltpu.sync_copy(tt.at[pl.ds(rbase, NCH)], idx_t)
        pltpu.sync_copy(rr.at[pl.ds(rbase, NCH)], idx_r)
        remap(idx_h)
        remap(idx_t)
        copies = []
        for j in range(NCH):
            sl = pl.ds(j * CHUNK, CHUNK)
            copies.append(pltpu.async_copy(ent1.at[idx_h.at[j]], e1h.at[sl], sem))
            copies.append(pltpu.async_copy(ent2.at[idx_h.at[j]], e2h.at[sl], sem))
            copies.append(pltpu.async_copy(ent1.at[idx_t.at[j]], e1t.at[sl], sem))
            copies.append(pltpu.async_copy(ent2.at[idx_t.at[j]], e2t.at[sl], sem))
            copies.append(pltpu.async_copy(rel1.at[idx_r.at[j]], r1v.at[sl], sem))
            copies.append(pltpu.async_copy(rel2.at[idx_r.at[j]], r2v.at[sl], sem))
        for c in copies:
            c.wait()

        lane_ids = lax.iota(jnp.int32, 16)

        def group(g, acc):
            base = g * 16
            svec = jnp.zeros((16,), jnp.float32)
            for k in range(16):
                r = base + k
                s = jnp.zeros((16,), jnp.float32)
                for h0 in (0, 16):
                    a = e1h[r, pl.ds(h0, 16)]
                    b = e2h[r, pl.ds(h0, 16)]
                    c_ = e1t[r, pl.ds(h0, 16)]
                    d_ = e2t[r, pl.ds(h0, 16)]
                    p = r1v[r, pl.ds(h0, 16)]
                    q = r2v[r, pl.ds(h0, 16)]
                    s = s + (a * c_ + b * d_) * p + (a * d_ - b * c_) * q
                    acc = acc + (a * a + b * b) + (c_ * c_ + d_ * d_) + (p * p + q * q)
                svec = jnp.where(lane_ids == k, jnp.sum(s), svec)
            score_v[pl.ds(base, 16)] = svec
            return acc

        total_sq = total_sq + lax.fori_loop(0, BPW // 16, group, jnp.zeros((16,), jnp.float32))
        pltpu.sync_copy(score_v, out_ref.at[pl.ds(sbase, BPW)])

    sq_v[...] = total_sq
    pltpu.sync_copy(sq_v, sq_out.at[wid])


_SEG = 1953 * 128
_RGRID = 1954
_RROWS_OUT = _RGRID * 128
_ROUT = 250064
_GTAB = _ROUT * 4


def _tc_repack_body(x0_ref, x1_ref, x2_ref, x3_ref, o_ref):
    o_ref[...] = jnp.concatenate(
        [x0_ref[...].T, x1_ref[...].T, x2_ref[...].T, x3_ref[...].T], axis=1)


_tc_repack = pl.pallas_call(
    _tc_repack_body,
    grid=(_RGRID,),
    in_specs=[
        pl.BlockSpec((D, 128), lambda i, j=j: (0, j * 1953 + i))
        for j in range(4)
    ],
    out_specs=pl.BlockSpec((128, 128), lambda i: (i, 0)),
    out_shape=jax.ShapeDtypeStruct((_ROUT, 128), jnp.float32),
)


def _tc_finish_body(ps_ref, ns_ref, py_ref, ny_ref, sq_ref, o_ref):
    x = -py_ref[...] * ps_ref[...]
    y = -ny_ref[...] * ns_ref[...]
    sp = jnp.maximum(x, 0.0) + jnp.log1p(jnp.exp(-jnp.abs(x)))
    sn = jnp.maximum(y, 0.0) + jnp.log1p(jnp.exp(-jnp.abs(y)))
    loss = (jnp.sum(sp) + jnp.sum(sn)) / B
    reg = jnp.sum(sq_ref[...]) / (B * D)
    o_ref[0, 0] = loss + LMBDA * reg


_tc_finish = pl.pallas_call(
    _tc_finish_body,
    out_shape=jax.ShapeDtypeStruct((1, 1), jnp.float32),
    out_specs=pl.BlockSpec(memory_space=pltpu.SMEM),
)


def kernel(pos_h, pos_t, pos_r, neg_h, neg_t, neg_r, pos_y, neg_y,
           ent1, ent2, rel1, rel2):
    ph = pos_h.astype(jnp.int32).reshape(NW * NCH, CHUNK)
    pt = pos_t.astype(jnp.int32).reshape(NW * NCH, CHUNK)
    pr = pos_r.astype(jnp.int32).reshape(NW * NCH, CHUNK)
    nh = neg_h.astype(jnp.int32).reshape(NW * NCH, CHUNK)
    nt = neg_t.astype(jnp.int32).reshape(NW * NCH, CHUNK)
    nr = neg_r.astype(jnp.int32).reshape(NW * NCH, CHUNK)
    e1t = ent1.T
    e2t = ent2.T
    ent1p = _tc_repack(e1t, e1t, e1t, e1t).reshape(_GTAB * D).reshape(_GTAB, D)
    ent2p = _tc_repack(e2t, e2t, e2t, e2t).reshape(_GTAB * D).reshape(_GTAB, D)
    ps, ns, sq = _sc_score(ph, pt, pr, nh, nt, nr, ent1p, ent2p, rel1, rel2)
    out = _tc_finish(ps.reshape(128, 128), ns.reshape(128, 128),
                     pos_y.reshape(128, 128), neg_y.reshape(128, 128), sq)
    return out[0, 0]


# speedup vs baseline: 4.3415x; 4.3415x over previous
import functools

import jax
import jax.numpy as jnp
from jax import lax
from jax.experimental import pallas as pl
from jax.experimental.pallas import tpu as pltpu
from jax.experimental.pallas import tpu_sc as plsc

ENT = 1000000
REL = 1000
D = 32
B = 16384
LMBDA = 0.0001

NC = 2
NS = 16
NW = NC * NS
BPW = B // NW
CHUNK = 128
NCH = BPW // CHUNK

_mesh = plsc.VectorSubcoreMesh(core_axis_name="c", subcore_axis_name="s")


@functools.partial(
    pl.kernel,
    mesh=_mesh,
    compiler_params=pltpu.CompilerParams(
        needs_layout_passes=False, use_tc_tiling_on_sc=False),
    out_type=[
        jax.ShapeDtypeStruct((B,), jnp.float32),
        jax.ShapeDtypeStruct((B,), jnp.float32),
        jax.ShapeDtypeStruct((NW, 16), jnp.float32),
    ],
    scratch_types=[
        pltpu.VMEM((NCH, CHUNK), jnp.int32),
        pltpu.VMEM((NCH, CHUNK), jnp.int32),
        pltpu.VMEM((NCH, CHUNK), jnp.int32),
        pltpu.VMEM((BPW, D), jnp.float32),
        pltpu.VMEM((BPW, D), jnp.float32),
        pltpu.VMEM((BPW, D), jnp.float32),
        pltpu.VMEM((BPW, D), jnp.float32),
        pltpu.VMEM((BPW, D), jnp.float32),
        pltpu.VMEM((BPW, D), jnp.float32),
        pltpu.VMEM((BPW,), jnp.float32),
        pltpu.VMEM((16,), jnp.float32),
        pltpu.SemaphoreType.DMA,
    ],
)
def _sc_score(ph, pt, pr, nh, nt, nr, ent1, ent2, rel1, rel2,
              ps_out, ns_out, sq_out,
              idx_h, idx_t, idx_r, e1h, e2h, e1t, e2t, r1v, r2v,
              score_v, sq_v, sem):
    wid = lax.axis_index("s") * NC + lax.axis_index("c")
    rbase = wid * NCH
    sbase = wid * BPW

    def remap(buf):
        for j in range(NCH):
            for k in range(128 // 16):
                e = buf[j, pl.ds(k * 16, 16)]
                q = ((e >= _SEG).astype(jnp.int32)
                     + (e >= 2 * _SEG).astype(jnp.int32)
                     + (e >= 3 * _SEG).astype(jnp.int32))
                g = 4 * (e - q * _SEG) + q
                buf[j, pl.ds(k * 16, 16)] = jnp.where(
                    e >= 4 * _SEG, 4 * e - (12 * _SEG - 3), g)

    total_sq = jnp.zeros((16,), jnp.float32)
    for hh, tt, rr, out_ref in ((ph, pt, pr, ps_out), (nh, nt, nr, ns_out)):
        pltpu.sync_copy(hh.at[pl.ds(rbase, NCH)], idx_h)
        pltpu.sync_copy(tt.at[pl.ds(rbase, NCH)], idx_t)
        pltpu.sync_copy(rr.at[pl.ds(rbase, NCH)], idx_r)
        remap(idx_h)
        remap(idx_t)
        copies = []
        for j in range(NCH):
            sl = pl.ds(j * CHUNK, CHUNK)
            copies.append(pltpu.async_copy(ent1.at[idx_h.at[j]], e1h.at[sl], sem))
            copies.append(pltpu.async_copy(ent2.at[idx_h.at[j]], e2h.at[sl], sem))
            copies.append(pltpu.async_copy(ent1.at[idx_t.at[j]], e1t.at[sl], sem))
            copies.append(pltpu.async_copy(ent2.at[idx_t.at[j]], e2t.at[sl], sem))
            copies.append(pltpu.async_copy(rel1.at[idx_r.at[j]], r1v.at[sl], sem))
            copies.append(pltpu.async_copy(rel2.at[idx_r.at[j]], r2v.at[sl], sem))
        for c in copies:
            c.wait()

        lane_ids = lax.iota(jnp.int32, 16)

        def group(g, acc):
            base = g * 16
            svec = jnp.zeros((16,), jnp.float32)
            for k in range(16):
                r = base + k
                s = jnp.zeros((16,), jnp.float32)
                for h0 in (0, 16):
                    a = e1h[r, pl.ds(h0, 16)]
                    b = e2h[r, pl.ds(h0, 16)]
                    c_ = e1t[r, pl.ds(h0, 16)]
                    d_ = e2t[r, pl.ds(h0, 16)]
                    p = r1v[r, pl.ds(h0, 16)]
                    q = r2v[r, pl.ds(h0, 16)]
                    s = s + (a * c_ + b * d_) * p + (a * d_ - b * c_) * q
                    acc = acc + (a * a + b * b) + (c_ * c_ + d_ * d_) + (p * p + q * q)
                svec = jnp.where(lane_ids == k, jnp.sum(s), svec)
            score_v[pl.ds(base, 16)] = svec
            return acc

        total_sq = total_sq + lax.fori_loop(0, BPW // 16, group, jnp.zeros((16,), jnp.float32))
        pltpu.sync_copy(score_v, out_ref.at[pl.ds(sbase, BPW)])

    sq_v[...] = total_sq
    pltpu.sync_copy(sq_v, sq_out.at[wid])


_SEG = 1953 * 128
_W = 3968
_NSTEP = _SEG // _W
_RGRID = _NSTEP + 1
_ROUT = _SEG + _W
_GTAB = _ROUT * 4


def _tc_repack_body(x0_ref, x1_ref, x2_ref, x3_ref, o_ref):
    eye = (lax.broadcasted_iota(jnp.int32, (D, D), 0)
           == lax.broadcasted_iota(jnp.int32, (D, D), 1)).astype(jnp.float32)
    cols = [lax.dot_general(x[...], eye, (((0,), (0,)), ((), ())),
                            preferred_element_type=jnp.float32)
            for x in (x0_ref, x1_ref, x2_ref, x3_ref)]
    o_ref[...] = jnp.concatenate(cols, axis=1)


_tc_repack = pl.pallas_call(
    _tc_repack_body,
    grid=(_RGRID,),
    in_specs=[
        pl.BlockSpec((D, _W), lambda i, j=j: (0, j * _NSTEP + i))
        for j in range(4)
    ],
    out_specs=pl.BlockSpec((_W, 128), lambda i: (i, 0)),
    out_shape=jax.ShapeDtypeStruct((_ROUT, 128), jnp.float32),
    compiler_params=pltpu.CompilerParams(fuse_transposed_lhs_in_matmul=True),
)


def _tc_finish_body(ps_ref, ns_ref, py_ref, ny_ref, sq_ref, o_ref):
    x = -py_ref[...] * ps_ref[...]
    y = -ny_ref[...] * ns_ref[...]
    sp = jnp.maximum(x, 0.0) + jnp.log1p(jnp.exp(-jnp.abs(x)))
    sn = jnp.maximum(y, 0.0) + jnp.log1p(jnp.exp(-jnp.abs(y)))
    loss = (jnp.sum(sp) + jnp.sum(sn)) / B
    reg = jnp.sum(sq_ref[...]) / (B * D)
    o_ref[0, 0] = loss + LMBDA * reg


_tc_finish = pl.pallas_call(
    _tc_finish_body,
    out_shape=jax.ShapeDtypeStruct((1, 1), jnp.float32),
    out_specs=pl.BlockSpec(memory_space=pltpu.SMEM),
)


def kernel(pos_h, pos_t, pos_r, neg_h, neg_t, neg_r, pos_y, neg_y,
           ent1, ent2, rel1, rel2):
    ph = pos_h.astype(jnp.int32).reshape(NW * NCH, CHUNK)
    pt = pos_t.astype(jnp.int32).reshape(NW * NCH, CHUNK)
    pr = pos_r.astype(jnp.int32).reshape(NW * NCH, CHUNK)
    nh = neg_h.astype(jnp.int32).reshape(NW * NCH, CHUNK)
    nt = neg_t.astype(jnp.int32).reshape(NW * NCH, CHUNK)
    nr = neg_r.astype(jnp.int32).reshape(NW * NCH, CHUNK)
    e1t = ent1.T
    e2t = ent2.T
    ent1p = _tc_repack(e1t, e1t, e1t, e1t).reshape(_GTAB * D).reshape(_GTAB, D)
    ent2p = _tc_repack(e2t, e2t, e2t, e2t).reshape(_GTAB * D).reshape(_GTAB, D)
    ps, ns, sq = _sc_score(ph, pt, pr, nh, nt, nr, ent1p, ent2p, rel1, rel2)
    out = _tc_finish(ps.reshape(128, 128), ns.reshape(128, 128),
                     pos_y.reshape(128, 128), neg_y.reshape(128, 128), sq)
    return out[0, 0]


# speedup vs baseline: 5.1301x; 1.1817x over previous
import functools

import jax
import jax.numpy as jnp
from jax import lax
from jax.experimental import pallas as pl
from jax.experimental.pallas import tpu as pltpu
from jax.experimental.pallas import tpu_sc as plsc

ENT = 1000000
REL = 1000
D = 32
B = 16384
LMBDA = 0.0001

NC = 2
NS = 16
NW = NC * NS
BPW = B // NW
NCH = BPW // 128

_CT = 7813
_WD = 601 * 128
_DROWS = 4 * _CT * 8
_GN = _DROWS * 128


def _tc_dump_body(x_ref, o_ref):
    for k in range(601):
        o_ref[8 * k:8 * (k + 1), :] = x_ref[:, 128 * k:128 * (k + 1)]


_tc_dump = pl.pallas_call(
    _tc_dump_body,
    grid=(4, 13),
    in_specs=[pl.BlockSpec((8, _WD), lambda g, i: (g, i))],
    out_specs=pl.BlockSpec((601 * 8, 128), lambda g, i: (g * 13 + i, 0)),
    out_shape=jax.ShapeDtypeStruct((_DROWS, 128), jnp.float32),
)

_mesh = plsc.VectorSubcoreMesh(core_axis_name="c", subcore_axis_name="s")


@functools.partial(
    pl.kernel,
    mesh=_mesh,
    compiler_params=pltpu.CompilerParams(
        needs_layout_passes=False, use_tc_tiling_on_sc=False),
    out_type=[
        jax.ShapeDtypeStruct((B,), jnp.float32),
        jax.ShapeDtypeStruct((B,), jnp.float32),
        jax.ShapeDtypeStruct((NW, 16), jnp.float32),
    ],
    scratch_types=[
        pltpu.VMEM((NCH, 128), jnp.int32),
        pltpu.VMEM((NCH, 128), jnp.int32),
        pltpu.VMEM((NCH, 128), jnp.int32),
        pltpu.VMEM((8, 128), jnp.int32),
        pltpu.VMEM((D, BPW), jnp.float32),
        pltpu.VMEM((D, BPW), jnp.float32),
        pltpu.VMEM((D, BPW), jnp.float32),
        pltpu.VMEM((D, BPW), jnp.float32),
        pltpu.VMEM((BPW, D), jnp.float32),
        pltpu.VMEM((BPW, D), jnp.float32),
        pltpu.VMEM((BPW,), jnp.float32),
        pltpu.VMEM((16,), jnp.float32),
        pltpu.SemaphoreType.DMA,
        pltpu.SemaphoreType.DMA,
    ],
)
def _sc_score(ph, pt, pr, nh, nt, nr, ent1d, ent2d, rel1, rel2,
              ps_out, ns_out, sq_out,
              idx_h, idx_t, idx_r, sidx, p1h, p2h, p1t, p2t, r1v, r2v,
              score_v, sq_v, sem_g, sem_r):
    wid = lax.axis_index("s") * NC + lax.axis_index("c")
    rbase = wid * NCH
    sbase = wid * BPW

    def to_base(buf):
        for j in range(NCH):
            for k in range(128 // 16):
                e = buf[j, pl.ds(k * 16, 16)]
                buf[j, pl.ds(k * 16, 16)] = ((e >> 7) << 10) + (e & 127)

    total_sq = jnp.zeros((16,), jnp.float32)
    for hh, tt, rr, out_ref in ((ph, pt, pr, ps_out), (nh, nt, nr, ns_out)):
        pltpu.sync_copy(hh.at[pl.ds(rbase, NCH)], idx_h)
        pltpu.sync_copy(tt.at[pl.ds(rbase, NCH)], idx_t)
        pltpu.sync_copy(rr.at[pl.ds(rbase, NCH)], idx_r)
        rel_copies = []
        for j in range(NCH):
            sl = pl.ds(j * 128, 128)
            rel_copies.append(
                pltpu.async_copy(rel1.at[idx_r.at[j]], r1v.at[sl], sem_r))
            rel_copies.append(
                pltpu.async_copy(rel2.at[idx_r.at[j]], r2v.at[sl], sem_r))
        to_base(idx_h)
        to_base(idx_t)

        def dim_round(d, carry):
            off = ((d >> 3) * 8000512 + (d & 7) * 128).astype(jnp.int32)
            for a, src in ((0, idx_h), (1, idx_t)):
                for j in range(NCH):
                    for k in range(128 // 16):
                        sidx[a * NCH + j, pl.ds(k * 16, 16)] = (
                            src[j, pl.ds(k * 16, 16)] + off)
            copies = []
            for a, d1, d2 in ((0, p1h, p2h), (1, p1t, p2t)):
                for j in range(NCH):
                    sl = pl.ds(j * 128, 128)
                    lref = sidx.at[a * NCH + j]
                    copies.append(
                        pltpu.async_copy(ent1d.at[lref], d1.at[d, sl], sem_g))
                    copies.append(
                        pltpu.async_copy(ent2d.at[lref], d2.at[d, sl], sem_g))
            for c in copies:
                c.wait()
            return carry

        lax.fori_loop(0, D, dim_round, 0)
        for c in rel_copies:
            c.wait()

        def vec_group(v, acc):
            ebase = v * 16
            rvec = idx_r[v >> 3, pl.ds((v & 7) * 16, 16)]
            svec = jnp.zeros((16,), jnp.float32)
            for d in range(D):
                a = p1h[d, pl.ds(ebase, 16)]
                b = p2h[d, pl.ds(ebase, 16)]
                c_ = p1t[d, pl.ds(ebase, 16)]
                d_ = p2t[d, pl.ds(ebase, 16)]
                dvec = jnp.full((16,), d, jnp.int32)
                p = plsc.load_gather(r1v, [rvec, dvec])
                q = plsc.load_gather(r2v, [rvec, dvec])
                svec = svec + (a * c_ + b * d_) * p + (a * d_ - b * c_) * q
                acc = acc + (a * a + b * b) + (c_ * c_ + d_ * d_) + (p * p + q * q)
            score_v[pl.ds(ebase, 16)] = svec
            return acc

        total_sq = total_sq + lax.fori_loop(
            0, BPW // 16, vec_group, jnp.zeros((16,), jnp.float32))
        pltpu.sync_copy(score_v, out_ref.at[pl.ds(sbase, BPW)])

    sq_v[...] = total_sq
    pltpu.sync_copy(sq_v, sq_out.at[wid])


def _tc_finish_body(ps_ref, ns_ref, py_ref, ny_ref, sq_ref, o_ref):
    x = -py_ref[...] * ps_ref[...]
    y = -ny_ref[...] * ns_ref[...]
    sp = jnp.maximum(x, 0.0) + jnp.log1p(jnp.exp(-jnp.abs(x)))
    sn = jnp.maximum(y, 0.0) + jnp.log1p(jnp.exp(-jnp.abs(y)))
    loss = (jnp.sum(sp) + jnp.sum(sn)) / B
    reg = jnp.sum(sq_ref[...]) / (B * D)
    o_ref[0, 0] = loss + LMBDA * reg


_tc_finish = pl.pallas_call(
    _tc_finish_body,
    out_shape=jax.ShapeDtypeStruct((1, 1), jnp.float32),
    out_specs=pl.BlockSpec(memory_space=pltpu.SMEM),
)


def kernel(pos_h, pos_t, pos_r, neg_h, neg_t, neg_r, pos_y, neg_y,
           ent1, ent2, rel1, rel2):
    ph = pos_h.astype(jnp.int32).reshape(NW * NCH, 128)
    pt = pos_t.astype(jnp.int32).reshape(NW * NCH, 128)
    pr = pos_r.astype(jnp.int32).reshape(NW * NCH, 128)
    nh = neg_h.astype(jnp.int32).reshape(NW * NCH, 128)
    nt = neg_t.astype(jnp.int32).reshape(NW * NCH, 128)
    nr = neg_r.astype(jnp.int32).reshape(NW * NCH, 128)
    ent1d = _tc_dump(ent1.T).reshape(_GN)
    ent2d = _tc_dump(ent2.T).reshape(_GN)
    ps, ns, sq = _sc_score(ph, pt, pr, nh, nt, nr, ent1d, ent2d, rel1, rel2)
    out = _tc_finish(ps.reshape(128, 128), ns.reshape(128, 128),
                     pos_y.reshape(128, 128), neg_y.reshape(128, 128), sq)
    return out[0, 0]


# speedup vs baseline: 5.2666x; 1.0266x over previous
import functools

import jax
import jax.numpy as jnp
from jax import lax
from jax.experimental import pallas as pl
from jax.experimental.pallas import tpu as pltpu
from jax.experimental.pallas import tpu_sc as plsc

ENT = 1000000
REL = 1000
D = 32
B = 16384
LMBDA = 0.0001

NC = 2
NS = 16
NW = NC * NS
BPW = B // NW
NCH = BPW // 128

_CT = 7813
_WD = 601 * 128
_DROWS = 4 * _CT * 8
_GN = _DROWS * 128


def _tc_dump_body(x1_ref, x2_ref, o1_ref, o2_ref):
    for k in range(601):
        o1_ref[8 * k:8 * (k + 1), :] = x1_ref[:, 128 * k:128 * (k + 1)]
        o2_ref[8 * k:8 * (k + 1), :] = x2_ref[:, 128 * k:128 * (k + 1)]


_tc_dump = pl.pallas_call(
    _tc_dump_body,
    grid=(4, 13),
    in_specs=[pl.BlockSpec((8, _WD), lambda g, i: (g, i))] * 2,
    out_specs=[pl.BlockSpec((601 * 8, 128), lambda g, i: (g * 13 + i, 0))] * 2,
    out_shape=[jax.ShapeDtypeStruct((_DROWS, 128), jnp.float32)] * 2,
)

_mesh = plsc.VectorSubcoreMesh(core_axis_name="c", subcore_axis_name="s")


@functools.partial(
    pl.kernel,
    mesh=_mesh,
    compiler_params=pltpu.CompilerParams(
        needs_layout_passes=False, use_tc_tiling_on_sc=False),
    out_type=[
        jax.ShapeDtypeStruct((B,), jnp.float32),
        jax.ShapeDtypeStruct((B,), jnp.float32),
        jax.ShapeDtypeStruct((NW, 16), jnp.float32),
    ],
    scratch_types=[
        pltpu.VMEM((NCH, 128), jnp.int32),
        pltpu.VMEM((NCH, 128), jnp.int32),
        pltpu.VMEM((NCH, 128), jnp.int32),
        pltpu.VMEM((2, BPW), jnp.int32),
        pltpu.VMEM((D, BPW), jnp.float32),
        pltpu.VMEM((D, BPW), jnp.float32),
        pltpu.VMEM((D, BPW), jnp.float32),
        pltpu.VMEM((D, BPW), jnp.float32),
        pltpu.VMEM((BPW, D), jnp.float32),
        pltpu.VMEM((BPW, D), jnp.float32),
        pltpu.VMEM((BPW,), jnp.float32),
        pltpu.VMEM((16,), jnp.float32),
        pltpu.SemaphoreType.DMA,
        pltpu.SemaphoreType.DMA,
    ],
)
def _sc_score(ph, pt, pr, nh, nt, nr, ent1d, ent2d, rel1, rel2,
              ps_out, ns_out, sq_out,
              idx_h, idx_t, idx_r, sidx, p1h, p2h, p1t, p2t, r1v, r2v,
              score_v, sq_v, sem_g, sem_r):
    wid = lax.axis_index("s") * NC + lax.axis_index("c")
    rbase = wid * NCH
    sbase = wid * BPW

    def to_base(buf):
        for j in range(NCH):
            for k in range(128 // 16):
                e = buf[j, pl.ds(k * 16, 16)]
                buf[j, pl.ds(k * 16, 16)] = ((e >> 7) << 10) + (e & 127)

    total_sq = jnp.zeros((16,), jnp.float32)
    for hh, tt, rr, out_ref in ((ph, pt, pr, ps_out), (nh, nt, nr, ns_out)):
        pltpu.sync_copy(hh.at[pl.ds(rbase, NCH)], idx_h)
        pltpu.sync_copy(tt.at[pl.ds(rbase, NCH)], idx_t)
        pltpu.sync_copy(rr.at[pl.ds(rbase, NCH)], idx_r)
        rel_copies = []
        for j in range(NCH):
            sl = pl.ds(j * 128, 128)
            rel_copies.append(
                pltpu.async_copy(rel1.at[idx_r.at[j]], r1v.at[sl], sem_r))
            rel_copies.append(
                pltpu.async_copy(rel2.at[idx_r.at[j]], r2v.at[sl], sem_r))
        to_base(idx_h)
        to_base(idx_t)

        def dim_round(d, carry):
            off = (d >> 3) * 8000512 + (d & 7) * 128
            for a, src in ((0, idx_h), (1, idx_t)):
                for j in range(NCH):
                    for k in range(128 // 16):
                        sidx[a, pl.ds(j * 128 + k * 16, 16)] = (
                            src[j, pl.ds(k * 16, 16)] + off)
            copies = []
            for a, d1, d2 in ((0, p1h, p2h), (1, p1t, p2t)):
                lref = sidx.at[a]
                copies.append(
                    pltpu.async_copy(ent1d.at[lref], d1.at[d], sem_g))
                copies.append(
                    pltpu.async_copy(ent2d.at[lref], d2.at[d], sem_g))
            for c in copies:
                c.wait()
            return carry

        lax.fori_loop(0, D, dim_round, 0)
        for c in rel_copies:
            c.wait()

        def vec_group(v, acc):
            ebase = v * 16
            rvec = idx_r[v >> 3, pl.ds((v & 7) * 16, 16)]
            svec = jnp.zeros((16,), jnp.float32)
            for d in range(D):
                a = p1h[d, pl.ds(ebase, 16)]
                b = p2h[d, pl.ds(ebase, 16)]
                c_ = p1t[d, pl.ds(ebase, 16)]
                d_ = p2t[d, pl.ds(ebase, 16)]
                dvec = jnp.full((16,), d, jnp.int32)
                p = plsc.load_gather(r1v, [rvec, dvec])
                q = plsc.load_gather(r2v, [rvec, dvec])
                svec = svec + (a * c_ + b * d_) * p + (a * d_ - b * c_) * q
                acc = acc + (a * a + b * b) + (c_ * c_ + d_ * d_) + (p * p + q * q)
            score_v[pl.ds(ebase, 16)] = svec
            return acc

        total_sq = total_sq + lax.fori_loop(
            0, BPW // 16, vec_group, jnp.zeros((16,), jnp.float32))
        pltpu.sync_copy(score_v, out_ref.at[pl.ds(sbase, BPW)])

    sq_v[...] = total_sq
    pltpu.sync_copy(sq_v, sq_out.at[wid])


def _tc_finish_body(ps_ref, ns_ref, py_ref, ny_ref, sq_ref, o_ref):
    x = -py_ref[...] * ps_ref[...]
    y = -ny_ref[...] * ns_ref[...]
    sp = jnp.maximum(x, 0.0) + jnp.log1p(jnp.exp(-jnp.abs(x)))
    sn = jnp.maximum(y, 0.0) + jnp.log1p(jnp.exp(-jnp.abs(y)))
    loss = (jnp.sum(sp) + jnp.sum(sn)) / B
    reg = jnp.sum(sq_ref[...]) / (B * D)
    o_ref[0, 0] = loss + LMBDA * reg


_tc_finish = pl.pallas_call(
    _tc_finish_body,
    out_shape=jax.ShapeDtypeStruct((1, 1), jnp.float32),
    out_specs=pl.BlockSpec(memory_space=pltpu.SMEM),
)


def kernel(pos_h, pos_t, pos_r, neg_h, neg_t, neg_r, pos_y, neg_y,
           ent1, ent2, rel1, rel2):
    ph = pos_h.astype(jnp.int32).reshape(NW * NCH, 128)
    pt = pos_t.astype(jnp.int32).reshape(NW * NCH, 128)
    pr = pos_r.astype(jnp.int32).reshape(NW * NCH, 128)
    nh = neg_h.astype(jnp.int32).reshape(NW * NCH, 128)
    nt = neg_t.astype(jnp.int32).reshape(NW * NCH, 128)
    nr = neg_r.astype(jnp.int32).reshape(NW * NCH, 128)
    d1, d2 = _tc_dump(ent1.T, ent2.T)
    ent1d = d1.reshape(_GN)
    ent2d = d2.reshape(_GN)
    ps, ns, sq = _sc_score(ph, pt, pr, nh, nt, nr, ent1d, ent2d, rel1, rel2)
    out = _tc_finish(ps.reshape(128, 128), ns.reshape(128, 128),
                     pos_y.reshape(128, 128), neg_y.reshape(128, 128), sq)
    return out[0, 0]


# speedup vs baseline: 5.5793x; 1.0594x over previous
import functools

import jax
import jax.numpy as jnp
from jax import lax
from jax.experimental import pallas as pl
from jax.experimental.pallas import tpu as pltpu
from jax.experimental.pallas import tpu_sc as plsc

ENT = 1000000
REL = 1000
D = 32
B = 16384
LMBDA = 0.0001

NC = 2
NS = 16
NW = NC * NS
BPW = B // NW
NCH = BPW // 128

_CT = 7813
_WD = 601 * 128
_DROWS = 4 * _CT * 8
_GN = _DROWS * 128


def _tc_dump_body(x1_ref, x2_ref, o1_ref, o2_ref):
    for k in range(601):
        o1_ref[8 * k:8 * (k + 1), :] = x1_ref[:, 128 * k:128 * (k + 1)]
        o2_ref[8 * k:8 * (k + 1), :] = x2_ref[:, 128 * k:128 * (k + 1)]


_tc_dump = pl.pallas_call(
    _tc_dump_body,
    grid=(4, 13),
    in_specs=[pl.BlockSpec((8, _WD), lambda g, i: (g, i))] * 2,
    out_specs=[pl.BlockSpec((601 * 8, 128), lambda g, i: (g * 13 + i, 0))] * 2,
    out_shape=[jax.ShapeDtypeStruct((_DROWS, 128), jnp.float32)] * 2,
)

_mesh = plsc.VectorSubcoreMesh(core_axis_name="c", subcore_axis_name="s")


@functools.partial(
    pl.kernel,
    mesh=_mesh,
    compiler_params=pltpu.CompilerParams(
        needs_layout_passes=False, use_tc_tiling_on_sc=False),
    out_type=[
        jax.ShapeDtypeStruct((B,), jnp.float32),
        jax.ShapeDtypeStruct((B,), jnp.float32),
        jax.ShapeDtypeStruct((NW, 16), jnp.float32),
    ],
    scratch_types=[
        pltpu.VMEM((NCH, 128), jnp.int32),
        pltpu.VMEM((NCH, 128), jnp.int32),
        pltpu.VMEM((NCH, 128), jnp.int32),
        pltpu.VMEM((4, BPW), jnp.int32),
        pltpu.VMEM((D, BPW), jnp.float32),
        pltpu.VMEM((D, BPW), jnp.float32),
        pltpu.VMEM((D, BPW), jnp.float32),
        pltpu.VMEM((D, BPW), jnp.float32),
        pltpu.VMEM((BPW, D), jnp.float32),
        pltpu.VMEM((BPW, D), jnp.float32),
        pltpu.VMEM((BPW,), jnp.float32),
        pltpu.VMEM((16,), jnp.float32),
        pltpu.SemaphoreType.DMA,
        pltpu.SemaphoreType.DMA,
    ],
)
def _sc_score(ph, pt, pr, nh, nt, nr, ent1d, ent2d, rel1, rel2,
              ps_out, ns_out, sq_out,
              idx_h, idx_t, idx_r, sidx, p1h, p2h, p1t, p2t, r1v, r2v,
              score_v, sq_v, sem_g, sem_r):
    wid = lax.axis_index("s") * NC + lax.axis_index("c")
    rbase = wid * NCH
    sbase = wid * BPW

    def to_base(buf):
        for j in range(NCH):
            for k in range(128 // 16):
                e = buf[j, pl.ds(k * 16, 16)]
                buf[j, pl.ds(k * 16, 16)] = ((e >> 7) << 10) + (e & 127)

    total_sq = jnp.zeros((16,), jnp.float32)
    for hh, tt, rr, out_ref in ((ph, pt, pr, ps_out), (nh, nt, nr, ns_out)):
        pltpu.sync_copy(hh.at[pl.ds(rbase, NCH)], idx_h)
        pltpu.sync_copy(tt.at[pl.ds(rbase, NCH)], idx_t)
        pltpu.sync_copy(rr.at[pl.ds(rbase, NCH)], idx_r)
        rel_copies = []
        for j in range(NCH):
            sl = pl.ds(j * 128, 128)
            rel_copies.append(
                pltpu.async_copy(rel1.at[idx_r.at[j]], r1v.at[sl], sem_r))
            rel_copies.append(
                pltpu.async_copy(rel2.at[idx_r.at[j]], r2v.at[sl], sem_r))
        to_base(idx_h)
        to_base(idx_t)

        def dim_pair(i, carry):
            copies = []
            for par in range(2):
                d = 2 * i + par
                off = (d >> 3) * 8000512 + (d & 7) * 128
                for a, src in ((0, idx_h), (1, idx_t)):
                    for j in range(NCH):
                        for k in range(128 // 16):
                            sidx[2 * par + a, pl.ds(j * 128 + k * 16, 16)] = (
                                src[j, pl.ds(k * 16, 16)] + off)
                for a, d1, d2 in ((0, p1h, p2h), (1, p1t, p2t)):
                    lref = sidx.at[2 * par + a]
                    copies.append(
                        pltpu.async_copy(ent1d.at[lref], d1.at[d], sem_g))
                    copies.append(
                        pltpu.async_copy(ent2d.at[lref], d2.at[d], sem_g))
            for c in copies:
                c.wait()
            return carry

        lax.fori_loop(0, D // 2, dim_pair, 0)
        for c in rel_copies:
            c.wait()

        def vec_group(v, acc):
            ebase = v * 16
            rvec = idx_r[v >> 3, pl.ds((v & 7) * 16, 16)]
            svec = jnp.zeros((16,), jnp.float32)
            for d in range(D):
                a = p1h[d, pl.ds(ebase, 16)]
                b = p2h[d, pl.ds(ebase, 16)]
                c_ = p1t[d, pl.ds(ebase, 16)]
                d_ = p2t[d, pl.ds(ebase, 16)]
                dvec = jnp.full((16,), d, jnp.int32)
                p = plsc.load_gather(r1v, [rvec, dvec])
                q = plsc.load_gather(r2v, [rvec, dvec])
                svec = svec + (a * c_ + b * d_) * p + (a * d_ - b * c_) * q
                acc = acc + (a * a + b * b) + (c_ * c_ + d_ * d_) + (p * p + q * q)
            score_v[pl.ds(ebase, 16)] = svec
            return acc

        total_sq = total_sq + lax.fori_loop(
            0, BPW // 16, vec_group, jnp.zeros((16,), jnp.float32))
        pltpu.sync_copy(score_v, out_ref.at[pl.ds(sbase, BPW)])

    sq_v[...] = total_sq
    pltpu.sync_copy(sq_v, sq_out.at[wid])


def _tc_finish_body(ps_ref, ns_ref, py_ref, ny_ref, sq_ref, o_ref):
    x = -py_ref[...] * ps_ref[...]
    y = -ny_ref[...] * ns_ref[...]
    sp = jnp.maximum(x, 0.0) + jnp.log1p(jnp.exp(-jnp.abs(x)))
    sn = jnp.maximum(y, 0.0) + jnp.log1p(jnp.exp(-jnp.abs(y)))
    loss = (jnp.sum(sp) + jnp.sum(sn)) / B
    reg = jnp.sum(sq_ref[...]) / (B * D)
    o_ref[0, 0] = loss + LMBDA * reg


_tc_finish = pl.pallas_call(
    _tc_finish_body,
    out_shape=jax.ShapeDtypeStruct((1, 1), jnp.float32),
    out_specs=pl.BlockSpec(memory_space=pltpu.SMEM),
)


def kernel(pos_h, pos_t, pos_r, neg_h, neg_t, neg_r, pos_y, neg_y,
           ent1, ent2, rel1, rel2):
    ph = pos_h.astype(jnp.int32).reshape(NW * NCH, 128)
    pt = pos_t.astype(jnp.int32).reshape(NW * NCH, 128)
    pr = pos_r.astype(jnp.int32).reshape(NW * NCH, 128)
    nh = neg_h.astype(jnp.int32).reshape(NW * NCH, 128)
    nt = neg_t.astype(jnp.int32).reshape(NW * NCH, 128)
    nr = neg_r.astype(jnp.int32).reshape(NW * NCH, 128)
    d1, d2 = _tc_dump(ent1.T, ent2.T)
    ent1d = d1.reshape(_GN)
    ent2d = d2.reshape(_GN)
    ps, ns, sq = _sc_score(ph, pt, pr, nh, nt, nr, ent1d, ent2d, rel1, rel2)
    out = _tc_finish(ps.reshape(128, 128), ns.reshape(128, 128),
                     pos_y.reshape(128, 128), neg_y.reshape(128, 128), sq)
    return out[0, 0]


# speedup vs baseline: 5.7330x; 1.0275x over previous
import functools

import jax
import jax.numpy as jnp
from jax import lax
from jax.experimental import pallas as pl
from jax.experimental.pallas import tpu as pltpu
from jax.experimental.pallas import tpu_sc as plsc

ENT = 1000000
REL = 1000
D = 32
B = 16384
LMBDA = 0.0001

NC = 2
NS = 16
NW = NC * NS
BPW = B // NW
NCH = BPW // 128

_CT = 7813
_WD = 601 * 128
_DROWS = 4 * _CT * 8
_GN = _DROWS * 128


def _tc_dump_body(x1_ref, x2_ref, o1_ref, o2_ref):
    for k in range(601):
        o1_ref[8 * k:8 * (k + 1), :] = x1_ref[:, 128 * k:128 * (k + 1)]
        o2_ref[8 * k:8 * (k + 1), :] = x2_ref[:, 128 * k:128 * (k + 1)]


_tc_dump = pl.pallas_call(
    _tc_dump_body,
    grid=(4, 13),
    in_specs=[pl.BlockSpec((8, _WD), lambda g, i: (g, i))] * 2,
    out_specs=[pl.BlockSpec((601 * 8, 128), lambda g, i: (g * 13 + i, 0))] * 2,
    out_shape=[jax.ShapeDtypeStruct((_DROWS, 128), jnp.float32)] * 2,
)

_mesh = plsc.VectorSubcoreMesh(core_axis_name="c", subcore_axis_name="s")


@functools.partial(
    pl.kernel,
    mesh=_mesh,
    compiler_params=pltpu.CompilerParams(
        needs_layout_passes=False, use_tc_tiling_on_sc=False),
    out_type=[
        jax.ShapeDtypeStruct((B,), jnp.float32),
        jax.ShapeDtypeStruct((B,), jnp.float32),
        jax.ShapeDtypeStruct((NW, 16), jnp.float32),
    ],
    scratch_types=[
        pltpu.VMEM((NCH, 128), jnp.int32),
        pltpu.VMEM((NCH, 128), jnp.int32),
        pltpu.VMEM((NCH, 128), jnp.int32),
        pltpu.VMEM((8, BPW), jnp.int32),
        pltpu.VMEM((D, BPW), jnp.float32),
        pltpu.VMEM((D, BPW), jnp.float32),
        pltpu.VMEM((D, BPW), jnp.float32),
        pltpu.VMEM((D, BPW), jnp.float32),
        pltpu.VMEM((BPW, D), jnp.float32),
        pltpu.VMEM((BPW, D), jnp.float32),
        pltpu.VMEM((BPW,), jnp.float32),
        pltpu.VMEM((16,), jnp.float32),
        pltpu.SemaphoreType.DMA,
        pltpu.SemaphoreType.DMA,
    ],
)
def _sc_score(ph, pt, pr, nh, nt, nr, ent1d, ent2d, rel1, rel2,
              ps_out, ns_out, sq_out,
              idx_h, idx_t, idx_r, sidx, p1h, p2h, p1t, p2t, r1v, r2v,
              score_v, sq_v, sem_g, sem_r):
    wid = lax.axis_index("s") * NC + lax.axis_index("c")
    rbase = wid * NCH
    sbase = wid * BPW

    def to_base(buf):
        for j in range(NCH):
            for k in range(128 // 16):
                e = buf[j, pl.ds(k * 16, 16)]
                buf[j, pl.ds(k * 16, 16)] = ((e >> 7) << 10) + (e & 127)

    total_sq = jnp.zeros((16,), jnp.float32)
    for hh, tt, rr, out_ref in ((ph, pt, pr, ps_out), (nh, nt, nr, ns_out)):
        pltpu.sync_copy(hh.at[pl.ds(rbase, NCH)], idx_h)
        pltpu.sync_copy(tt.at[pl.ds(rbase, NCH)], idx_t)
        pltpu.sync_copy(rr.at[pl.ds(rbase, NCH)], idx_r)
        rel_copies = []
        for j in range(NCH):
            sl = pl.ds(j * 128, 128)
            rel_copies.append(
                pltpu.async_copy(rel1.at[idx_r.at[j]], r1v.at[sl], sem_r))
            rel_copies.append(
                pltpu.async_copy(rel2.at[idx_r.at[j]], r2v.at[sl], sem_r))
        to_base(idx_h)
        to_base(idx_t)

        def dim_pair(i, carry):
            copies = []
            for par in range(4):
                d = 4 * i + par
                off = (d >> 3) * 8000512 + (d & 7) * 128
                for a, src in ((0, idx_h), (1, idx_t)):
                    for j in range(NCH):
                        for k in range(128 // 16):
                            sidx[2 * par + a, pl.ds(j * 128 + k * 16, 16)] = (
                                src[j, pl.ds(k * 16, 16)] + off)
                for a, d1, d2 in ((0, p1h, p2h), (1, p1t, p2t)):
                    lref = sidx.at[2 * par + a]
                    copies.append(
                        pltpu.async_copy(ent1d.at[lref], d1.at[d], sem_g))
                    copies.append(
                        pltpu.async_copy(ent2d.at[lref], d2.at[d], sem_g))
            for c in copies:
                c.wait()
            return carry

        lax.fori_loop(0, D // 4, dim_pair, 0)
        for c in rel_copies:
            c.wait()

        def vec_group(v, acc):
            ebase = v * 16
            rvec = idx_r[v >> 3, pl.ds((v & 7) * 16, 16)]
            svec = jnp.zeros((16,), jnp.float32)
            for d in range(D):
                a = p1h[d, pl.ds(ebase, 16)]
                b = p2h[d, pl.ds(ebase, 16)]
                c_ = p1t[d, pl.ds(ebase, 16)]
                d_ = p2t[d, pl.ds(ebase, 16)]
                dvec = jnp.full((16,), d, jnp.int32)
                p = plsc.load_gather(r1v, [rvec, dvec])
                q = plsc.load_gather(r2v, [rvec, dvec])
                svec = svec + (a * c_ + b * d_) * p + (a * d_ - b * c_) * q
                acc = acc + (a * a + b * b) + (c_ * c_ + d_ * d_) + (p * p + q * q)
            score_v[pl.ds(ebase, 16)] = svec
            return acc

        total_sq = total_sq + lax.fori_loop(
            0, BPW // 16, vec_group, jnp.zeros((16,), jnp.float32))
        pltpu.sync_copy(score_v, out_ref.at[pl.ds(sbase, BPW)])

    sq_v[...] = total_sq
    pltpu.sync_copy(sq_v, sq_out.at[wid])


def _tc_finish_body(ps_ref, ns_ref, py_ref, ny_ref, sq_ref, o_ref):
    x = -py_ref[...] * ps_ref[...]
    y = -ny_ref[...] * ns_ref[...]
    sp = jnp.maximum(x, 0.0) + jnp.log1p(jnp.exp(-jnp.abs(x)))
    sn = jnp.maximum(y, 0.0) + jnp.log1p(jnp.exp(-jnp.abs(y)))
    loss = (jnp.sum(sp) + jnp.sum(sn)) / B
    reg = jnp.sum(sq_ref[...]) / (B * D)
    o_ref[0, 0] = loss + LMBDA * reg


_tc_finish = pl.pallas_call(
    _tc_finish_body,
    out_shape=jax.ShapeDtypeStruct((1, 1), jnp.float32),
    out_specs=pl.BlockSpec(memory_space=pltpu.SMEM),
)


def kernel(pos_h, pos_t, pos_r, neg_h, neg_t, neg_r, pos_y, neg_y,
           ent1, ent2, rel1, rel2):
    ph = pos_h.astype(jnp.int32).reshape(NW * NCH, 128)
    pt = pos_t.astype(jnp.int32).reshape(NW * NCH, 128)
    pr = pos_r.astype(jnp.int32).reshape(NW * NCH, 128)
    nh = neg_h.astype(jnp.int32).reshape(NW * NCH, 128)
    nt = neg_t.astype(jnp.int32).reshape(NW * NCH, 128)
    nr = neg_r.astype(jnp.int32).reshape(NW * NCH, 128)
    d1, d2 = _tc_dump(ent1.T, ent2.T)
    ent1d = d1.reshape(_GN)
    ent2d = d2.reshape(_GN)
    ps, ns, sq = _sc_score(ph, pt, pr, nh, nt, nr, ent1d, ent2d, rel1, rel2)
    out = _tc_finish(ps.reshape(128, 128), ns.reshape(128, 128),
                     pos_y.reshape(128, 128), neg_y.reshape(128, 128), sq)
    return out[0, 0]


# speedup vs baseline: 5.8159x; 1.0145x over previous
import functools

import jax
import jax.numpy as jnp
from jax import lax
from jax.experimental import pallas as pl
from jax.experimental.pallas import tpu as pltpu
from jax.experimental.pallas import tpu_sc as plsc

ENT = 1000000
REL = 1000
D = 32
B = 16384
LMBDA = 0.0001

NC = 2
NS = 16
NW = NC * NS
BPW = B // NW
NCH = BPW // 128

_CT = 7813
_WD = 601 * 128
_DROWS = 4 * _CT * 8
_GN = _DROWS * 128


def _tc_dump_body(x1_ref, x2_ref, o1_ref, o2_ref):
    for k in range(601):
        o1_ref[8 * k:8 * (k + 1), :] = x1_ref[:, 128 * k:128 * (k + 1)]
        o2_ref[8 * k:8 * (k + 1), :] = x2_ref[:, 128 * k:128 * (k + 1)]


_tc_dump = pl.pallas_call(
    _tc_dump_body,
    grid=(4, 13),
    in_specs=[pl.BlockSpec((8, _WD), lambda g, i: (g, i))] * 2,
    out_specs=[pl.BlockSpec((601 * 8, 128), lambda g, i: (g * 13 + i, 0))] * 2,
    out_shape=[jax.ShapeDtypeStruct((_DROWS, 128), jnp.float32)] * 2,
)

_mesh = plsc.VectorSubcoreMesh(core_axis_name="c", subcore_axis_name="s")


@functools.partial(
    pl.kernel,
    mesh=_mesh,
    compiler_params=pltpu.CompilerParams(
        needs_layout_passes=False, use_tc_tiling_on_sc=False),
    out_type=[
        jax.ShapeDtypeStruct((B,), jnp.float32),
        jax.ShapeDtypeStruct((B,), jnp.float32),
        jax.ShapeDtypeStruct((NW, 16), jnp.float32),
    ],
    scratch_types=[
        pltpu.VMEM((NCH, 128), jnp.int32),
        pltpu.VMEM((NCH, 128), jnp.int32),
        pltpu.VMEM((NCH, 128), jnp.int32),
        pltpu.VMEM((16, BPW), jnp.int32),
        pltpu.VMEM((D, BPW), jnp.float32),
        pltpu.VMEM((D, BPW), jnp.float32),
        pltpu.VMEM((D, BPW), jnp.float32),
        pltpu.VMEM((D, BPW), jnp.float32),
        pltpu.VMEM((BPW, D), jnp.float32),
        pltpu.VMEM((BPW, D), jnp.float32),
        pltpu.VMEM((BPW,), jnp.float32),
        pltpu.VMEM((16,), jnp.float32),
        pltpu.SemaphoreType.DMA,
        pltpu.SemaphoreType.DMA,
    ],
)
def _sc_score(ph, pt, pr, nh, nt, nr, ent1d, ent2d, rel1, rel2,
              ps_out, ns_out, sq_out,
              idx_h, idx_t, idx_r, sidx, p1h, p2h, p1t, p2t, r1v, r2v,
              score_v, sq_v, sem_g, sem_r):
    wid = lax.axis_index("s") * NC + lax.axis_index("c")
    rbase = wid * NCH
    sbase = wid * BPW

    def to_base(buf):
        for j in range(NCH):
            for k in range(128 // 16):
                e = buf[j, pl.ds(k * 16, 16)]
                buf[j, pl.ds(k * 16, 16)] = ((e >> 7) << 10) + (e & 127)

    total_sq = jnp.zeros((16,), jnp.float32)
    for hh, tt, rr, out_ref in ((ph, pt, pr, ps_out), (nh, nt, nr, ns_out)):
        pltpu.sync_copy(hh.at[pl.ds(rbase, NCH)], idx_h)
        pltpu.sync_copy(tt.at[pl.ds(rbase, NCH)], idx_t)
        pltpu.sync_copy(rr.at[pl.ds(rbase, NCH)], idx_r)
        rel_copies = []
        for j in range(NCH):
            sl = pl.ds(j * 128, 128)
            rel_copies.append(
                pltpu.async_copy(rel1.at[idx_r.at[j]], r1v.at[sl], sem_r))
            rel_copies.append(
                pltpu.async_copy(rel2.at[idx_r.at[j]], r2v.at[sl], sem_r))
        to_base(idx_h)
        to_base(idx_t)

        def dim_pair(i, carry):
            copies = []
            for par in range(8):
                d = 8 * i + par
                off = (d >> 3) * 8000512 + (d & 7) * 128
                for a, src in ((0, idx_h), (1, idx_t)):
                    for j in range(NCH):
                        for k in range(128 // 16):
                            sidx[2 * par + a, pl.ds(j * 128 + k * 16, 16)] = (
                                src[j, pl.ds(k * 16, 16)] + off)
                for a, d1, d2 in ((0, p1h, p2h), (1, p1t, p2t)):
                    lref = sidx.at[2 * par + a]
                    copies.append(
                        pltpu.async_copy(ent1d.at[lref], d1.at[d], sem_g))
                    copies.append(
                        pltpu.async_copy(ent2d.at[lref], d2.at[d], sem_g))
            for c in copies:
                c.wait()
            return carry

        lax.fori_loop(0, D // 8, dim_pair, 0)
        for c in rel_copies:
            c.wait()

        def vec_group(v, acc):
            ebase = v * 16
            rvec = idx_r[v >> 3, pl.ds((v & 7) * 16, 16)]
            svec = jnp.zeros((16,), jnp.float32)
            for d in range(D):
                a = p1h[d, pl.ds(ebase, 16)]
                b = p2h[d, pl.ds(ebase, 16)]
                c_ = p1t[d, pl.ds(ebase, 16)]
                d_ = p2t[d, pl.ds(ebase, 16)]
                dvec = jnp.full((16,), d, jnp.int32)
                p = plsc.load_gather(r1v, [rvec, dvec])
                q = plsc.load_gather(r2v, [rvec, dvec])
                svec = svec + (a * c_ + b * d_) * p + (a * d_ - b * c_) * q
                acc = acc + (a * a + b * b) + (c_ * c_ + d_ * d_) + (p * p + q * q)
            score_v[pl.ds(ebase, 16)] = svec
            return acc

        total_sq = total_sq + lax.fori_loop(
            0, BPW // 16, vec_group, jnp.zeros((16,), jnp.float32))
        pltpu.sync_copy(score_v, out_ref.at[pl.ds(sbase, BPW)])

    sq_v[...] = total_sq
    pltpu.sync_copy(sq_v, sq_out.at[wid])


def _tc_finish_body(ps_ref, ns_ref, py_ref, ny_ref, sq_ref, o_ref):
    x = -py_ref[...] * ps_ref[...]
    y = -ny_ref[...] * ns_ref[...]
    sp = jnp.maximum(x, 0.0) + jnp.log1p(jnp.exp(-jnp.abs(x)))
    sn = jnp.maximum(y, 0.0) + jnp.log1p(jnp.exp(-jnp.abs(y)))
    loss = (jnp.sum(sp) + jnp.sum(sn)) / B
    reg = jnp.sum(sq_ref[...]) / (B * D)
    o_ref[0, 0] = loss + LMBDA * reg


_tc_finish = pl.pallas_call(
    _tc_finish_body,
    out_shape=jax.ShapeDtypeStruct((1, 1), jnp.float32),
    out_specs=pl.BlockSpec(memory_space=pltpu.SMEM),
)


def kernel(pos_h, pos_t, pos_r, neg_h, neg_t, neg_r, pos_y, neg_y,
           ent1, ent2, rel1, rel2):
    ph = pos_h.astype(jnp.int32).reshape(NW * NCH, 128)
    pt = pos_t.astype(jnp.int32).reshape(NW * NCH, 128)
    pr = pos_r.astype(jnp.int32).reshape(NW * NCH, 128)
    nh = neg_h.astype(jnp.int32).reshape(NW * NCH, 128)
    nt = neg_t.astype(jnp.int32).reshape(NW * NCH, 128)
    nr = neg_r.astype(jnp.int32).reshape(NW * NCH, 128)
    d1, d2 = _tc_dump(ent1.T, ent2.T)
    ent1d = d1.reshape(_GN)
    ent2d = d2.reshape(_GN)
    ps, ns, sq = _sc_score(ph, pt, pr, nh, nt, nr, ent1d, ent2d, rel1, rel2)
    out = _tc_finish(ps.reshape(128, 128), ns.reshape(128, 128),
                     pos_y.reshape(128, 128), neg_y.reshape(128, 128), sq)
    return out[0, 0]
